# Initial kernel scaffold; baseline (speedup 1.0000x reference)
import functools

import jax
import jax.numpy as jnp
from jax import lax
from jax.experimental import pallas as pl
from jax.experimental.pallas import tpu as pltpu
from jax.experimental.pallas import tpu_sc as plsc

VOCAB = 1000
NC, NS = 2, 16
NW = NC * NS
ROWS = 1024 * 50
R_PER_W = ROWS // NW
CHUNK = 40
NCHUNK = R_PER_W // CHUNK


def _make_kernel():
  mesh = plsc.VectorSubcoreMesh(core_axis_name="c", subcore_axis_name="s")

  @functools.partial(
      pl.kernel,
      mesh=mesh,
      out_type=jax.ShapeDtypeStruct((ROWS, VOCAB), jnp.float32),
      scratch_types=[
          pltpu.VMEM((NCHUNK, CHUNK), jnp.int32),
          pltpu.VMEM((CHUNK, VOCAB), jnp.float32),
          pltpu.VMEM((CHUNK, VOCAB), jnp.float32),
          pltpu.SemaphoreType.DMA,
          pltpu.SemaphoreType.DMA,
      ],
  )
  def gather_kernel(table_hbm, idx_hbm, out_hbm, idx_v, buf0, buf1, sem0,
                    sem1):
    wid = lax.axis_index("s") * NC + lax.axis_index("c")
    base = wid * R_PER_W
    pltpu.sync_copy(idx_hbm.at[wid], idx_v)

    def gather(g, buf, sem):
      return pltpu.make_async_copy(table_hbm.at[idx_v.at[g]], buf, sem)

    def drain(g, buf):
      pltpu.sync_copy(buf, out_hbm.at[pl.ds(base + g * CHUNK, CHUNK)])

    gather(0, buf0, sem0).start()

    def body(i, carry):
      g0 = 2 * i
      g1 = g0 + 1
      gather(g1, buf1, sem1).start()
      gather(g0, buf0, sem0).wait()
      drain(g0, buf0)

      @pl.when(i < NCHUNK // 2 - 1)
      def _():
        gather(g1 + 1, buf0, sem0).start()

      gather(g1, buf1, sem1).wait()
      drain(g1, buf1)
      return carry

    lax.fori_loop(0, NCHUNK // 2, body, 0)

  return gather_kernel


_sc_gather = _make_kernel()


@jax.jit
def kernel(idx, table):
  flat_idx = idx.astype(jnp.int32).reshape(NW, NCHUNK, CHUNK)
  out = _sc_gather(table, flat_idx)
  return out.reshape(idx.shape[0], idx.shape[1], VOCAB)


# speedup vs baseline: 1.0345x; 1.0345x over previous
import functools

import jax
import jax.numpy as jnp
from jax import lax
from jax.experimental import pallas as pl
from jax.experimental.pallas import tpu as pltpu
from jax.experimental.pallas import tpu_sc as plsc

VOCAB = 1000
NC, NS = 2, 16
NW = NC * NS
ROWS = 1024 * 50
R_PER_W = ROWS // NW
CHUNK = 40
NCHUNK = R_PER_W // CHUNK


def _make_kernel():
  mesh = plsc.VectorSubcoreMesh(core_axis_name="c", subcore_axis_name="s")

  @functools.partial(
      pl.kernel,
      mesh=mesh,
      out_type=jax.ShapeDtypeStruct((ROWS, VOCAB), jnp.float32),
      scratch_types=[
          pltpu.VMEM((NCHUNK, CHUNK), jnp.int32),
          pltpu.VMEM((CHUNK, VOCAB), jnp.float32),
          pltpu.VMEM((CHUNK, VOCAB), jnp.float32),
          pltpu.SemaphoreType.DMA,
          pltpu.SemaphoreType.DMA,
      ],
      compiler_params=pltpu.CompilerParams(use_tc_tiling_on_sc=False),
  )
  def gather_kernel(table_hbm, idx_hbm, out_hbm, idx_v, buf0, buf1, sem0,
                    sem1):
    wid = lax.axis_index("s") * NC + lax.axis_index("c")
    base = wid * R_PER_W
    pltpu.sync_copy(idx_hbm.at[wid], idx_v)

    def gather(g, buf, sem):
      return pltpu.make_async_copy(table_hbm.at[idx_v.at[g]], buf, sem)

    def drain(g, buf):
      pltpu.sync_copy(buf, out_hbm.at[pl.ds(base + g * CHUNK, CHUNK)])

    gather(0, buf0, sem0).start()

    def body(i, carry):
      g0 = 2 * i
      g1 = g0 + 1
      gather(g1, buf1, sem1).start()
      gather(g0, buf0, sem0).wait()
      drain(g0, buf0)

      @pl.when(i < NCHUNK // 2 - 1)
      def _():
        gather(g1 + 1, buf0, sem0).start()

      gather(g1, buf1, sem1).wait()
      drain(g1, buf1)
      return carry

    lax.fori_loop(0, NCHUNK // 2, body, 0)

  return gather_kernel


_sc_gather = _make_kernel()


@jax.jit
def kernel(idx, table):
  flat_idx = idx.astype(jnp.int32).reshape(NW, NCHUNK, CHUNK)
  out = _sc_gather(table, flat_idx)
  return out.reshape(idx.shape[0], idx.shape[1], VOCAB)


# speedup vs baseline: 1.0354x; 1.0009x over previous
import functools

import jax
import jax.numpy as jnp
from jax import lax
from jax.experimental import pallas as pl
from jax.experimental.pallas import tpu as pltpu
from jax.experimental.pallas import tpu_sc as plsc

VOCAB = 1000
NC, NS = 2, 16
NW = NC * NS
ROWS = 1024 * 50
R_PER_W = ROWS // NW
CHUNK = 40
NCHUNK = R_PER_W // CHUNK
VPAD = 1024


def _make_kernel():
  mesh = plsc.VectorSubcoreMesh(core_axis_name="c", subcore_axis_name="s")

  @functools.partial(
      pl.kernel,
      mesh=mesh,
      out_type=jax.ShapeDtypeStruct((ROWS, VOCAB), jnp.float32),
      scratch_types=[
          pltpu.VMEM((NCHUNK, CHUNK), jnp.int32),
          pltpu.VMEM((CHUNK, VPAD), jnp.float32),
          pltpu.VMEM((CHUNK, VPAD), jnp.float32),
          pltpu.SemaphoreType.DMA,
          pltpu.SemaphoreType.DMA,
      ],
      compiler_params=pltpu.CompilerParams(use_tc_tiling_on_sc=False),
  )
  def gather_kernel(table_hbm, idx_hbm, out_hbm, idx_v, buf0, buf1, sem0,
                    sem1):
    wid = lax.axis_index("s") * NC + lax.axis_index("c")
    base = wid * R_PER_W
    pltpu.sync_copy(idx_hbm.at[wid], idx_v)

    def gather(g, buf, sem):
      return pltpu.make_async_copy(table_hbm.at[idx_v.at[g]], buf, sem)

    def drain(g, buf):
      pltpu.sync_copy(buf.at[:, pl.ds(0, VOCAB)],
                      out_hbm.at[pl.ds(base + g * CHUNK, CHUNK)])

    gather(0, buf0, sem0).start()

    def body(i, carry):
      g0 = 2 * i
      g1 = g0 + 1
      gather(g1, buf1, sem1).start()
      gather(g0, buf0, sem0).wait()
      drain(g0, buf0)

      @pl.when(i < NCHUNK // 2 - 1)
      def _():
        gather(g1 + 1, buf0, sem0).start()

      gather(g1, buf1, sem1).wait()
      drain(g1, buf1)
      return carry

    lax.fori_loop(0, NCHUNK // 2, body, 0)

  return gather_kernel


_sc_gather = _make_kernel()


@jax.jit
def kernel(idx, table):
  flat_idx = idx.astype(jnp.int32).reshape(NW, NCHUNK, CHUNK)
  table_p = jnp.pad(table, ((0, 0), (0, VPAD - VOCAB)))
  out = _sc_gather(table_p, flat_idx)
  return out.reshape(idx.shape[0], idx.shape[1], VOCAB)


# speedup vs baseline: 1.4193x; 1.3708x over previous
import functools

import jax
import jax.numpy as jnp
from jax import lax
from jax.experimental import pallas as pl
from jax.experimental.pallas import tpu as pltpu
from jax.experimental.pallas import tpu_sc as plsc
from jax.experimental import layout as jex_layout

VOCAB = 1000
NC, NS = 2, 16
NW = NC * NS
ROWS = 1024 * 50
R_PER_W = ROWS // NW
CHUNK = 40
NCHUNK = R_PER_W // CHUNK
VPAD = 1024


def _make_kernel():
  mesh = plsc.VectorSubcoreMesh(core_axis_name="c", subcore_axis_name="s",
                                num_cores=NC, num_subcores=NS)

  @functools.partial(
      pl.kernel,
      mesh=mesh,
      out_type=jax.ShapeDtypeStruct((ROWS, VPAD), jnp.float32),
      scratch_types=[
          pltpu.VMEM((NCHUNK, CHUNK), jnp.int32),
          pltpu.VMEM((CHUNK, VPAD), jnp.float32),
          pltpu.VMEM((CHUNK, VPAD), jnp.float32),
          pltpu.SemaphoreType.DMA,
          pltpu.SemaphoreType.DMA,
      ],
      compiler_params=pltpu.CompilerParams(use_tc_tiling_on_sc=True),
  )
  def gather_kernel(table_hbm, idx_hbm, out_hbm, idx_v, buf0, buf1, sem0,
                    sem1):
    wid = lax.axis_index("s") * NC + lax.axis_index("c")
    base = wid * R_PER_W
    pltpu.sync_copy(idx_hbm.at[wid], idx_v)

    def gather(g, buf, sem):
      return pltpu.make_async_copy(table_hbm.at[idx_v.at[g]], buf, sem)

    def drain(g, buf):
      pltpu.sync_copy(buf, out_hbm.at[pl.ds(base + g * CHUNK, CHUNK)])

    gather(0, buf0, sem0).start()

    def body(i, carry):
      g0 = 2 * i
      g1 = g0 + 1
      gather(g1, buf1, sem1).start()
      gather(g0, buf0, sem0).wait()
      drain(g0, buf0)

      @pl.when(i < NCHUNK // 2 - 1)
      def _():
        gather(g1 + 1, buf0, sem0).start()

      gather(g1, buf1, sem1).wait()
      drain(g1, buf1)
      return carry

    lax.fori_loop(0, NCHUNK // 2, body, 0)

  return gather_kernel


_sc_gather = _make_kernel()


def _impl(idx, table):
  flat_idx = idx.astype(jnp.int32).reshape(NW, NCHUNK, CHUNK)
  table_p = jnp.pad(table, ((0, 0), (0, VPAD - VOCAB)))
  out = _sc_gather(table_p, flat_idx)
  return out[:, :VOCAB].reshape(idx.shape[0], idx.shape[1], VOCAB)


kernel = jax.jit(_impl)
